# Initial kernel scaffold; baseline (speedup 1.0000x reference)
import jax
import jax.numpy as jnp
from jax.experimental import pallas as pl


def kernel(x, edge_index, mw1, mb1, mw2, mb2, mw3, mb3, cw1, cb1, cw2, cb2, cw3, cb3, cw4, cb4):
    raise NotImplementedError("write your pallas kernel here")


# speedup vs baseline: 2.4837x; 2.4837x over previous
import jax
import jax.numpy as jnp
from jax import lax
from jax.experimental import pallas as pl
from jax.experimental.pallas import tpu as pltpu
from jax.experimental.pallas import tpu_sc as plsc

N = 10000
E = 320000
D = 128
H = 128

_NC = 2
_NS = 16
_NW = _NC * _NS

_PW = E // _NW
_GC = 80
_RPT = 624
_RTAIL = N - _NS * _RPT

_BE = 2000
_BN = 1000


def _prep_body(x_ref, w_ref, b_ref, t_ref):
    xb = x_ref[...]
    t_ref[...] = jnp.dot(xb, w_ref[...], preferred_element_type=jnp.float32) + b_ref[...]


def _prep(x, w1a, mb1):
    return pl.pallas_call(
        _prep_body,
        grid=(N // _BN,),
        in_specs=[
            pl.BlockSpec((_BN, D), lambda i: (i, 0)),
            pl.BlockSpec((D, H), lambda i: (0, 0)),
            pl.BlockSpec((1, H), lambda i: (0, 0)),
        ],
        out_specs=pl.BlockSpec((_BN, H), lambda i: (i, 0)),
        out_shape=jax.ShapeDtypeStruct((N, H), jnp.float32),
    )(x, w1a, mb1)


def _gather_body(t_hbm, xtab_hbm, ytab_hbm, ztab_hbm, row_hbm, col_hbm,
                 tr_hbm, rel_hbm,
                 xtab_v, ytab_v, ztab_v, idxr_v, idxc_v, tr_v, rel_v, sem1):
    wid = lax.axis_index("s") * _NC + lax.axis_index("c")
    base0 = wid * _PW
    pltpu.sync_copy(xtab_hbm, xtab_v)
    pltpu.sync_copy(ytab_hbm, ytab_v)
    pltpu.sync_copy(ztab_hbm, ztab_v)

    def body(k, carry):
        base = base0 + k * _GC
        pltpu.sync_copy(row_hbm.at[pl.ds(base, _GC)], idxr_v)
        pltpu.sync_copy(col_hbm.at[pl.ds(base, _GC)], idxc_v)
        c1 = pltpu.async_copy(t_hbm.at[idxr_v], tr_v, sem1)
        for j in range(_GC // 16):
            ir = idxr_v[pl.ds(j * 16, 16)]
            ic = idxc_v[pl.ds(j * 16, 16)]
            erow = lax.iota(jnp.int32, 16) + j * 16
            for f, tab in enumerate((xtab_v, ytab_v, ztab_v)):
                r = plsc.load_gather(tab, [ic]) - plsc.load_gather(tab, [ir])
                plsc.store_scatter(rel_v, [erow, jnp.full((16,), f, jnp.int32)], r)
        c1.wait()
        pltpu.sync_copy(tr_v, tr_hbm.at[pl.ds(base, _GC)])
        pltpu.sync_copy(rel_v, rel_hbm.at[pl.ds(base, _GC)])
        return carry

    lax.fori_loop(0, _PW // _GC, body, 0)


_gather = pl.kernel(
    _gather_body,
    out_type=(
        jax.ShapeDtypeStruct((E, H), jnp.float32),
        jax.ShapeDtypeStruct((E, 8), jnp.float32),
    ),
    mesh=plsc.VectorSubcoreMesh(core_axis_name="c", subcore_axis_name="s"),
    compiler_params=pltpu.CompilerParams(needs_layout_passes=False),
    scratch_types=[
        pltpu.VMEM((N,), jnp.float32),
        pltpu.VMEM((N,), jnp.float32),
        pltpu.VMEM((N,), jnp.float32),
        pltpu.VMEM((_GC,), jnp.int32),
        pltpu.VMEM((_GC,), jnp.int32),
        pltpu.VMEM((_GC, H), jnp.float32),
        pltpu.VMEM((_GC, 8), jnp.float32),
        pltpu.SemaphoreType.DMA,
    ],
)


def _edge_body(tr_ref, rel_ref, wrel_ref, wdist_ref, wunit_ref,
               w2_ref, b2_ref, w3_ref, b3_ref, out_ref):
    lane = lax.broadcasted_iota(jnp.int32, (_BE, 8), 1)
    rel = jnp.where(lane < 3, rel_ref[...], 0.0)
    d2 = jnp.sum(rel * rel, axis=1, keepdims=True)
    dist = jnp.sqrt(d2) + 1e-12
    unit = rel / dist
    h = (tr_ref[...]
         + jnp.dot(rel, wrel_ref[...], preferred_element_type=jnp.float32)
         + jnp.dot(unit, wunit_ref[...], preferred_element_type=jnp.float32)
         + dist * wdist_ref[...])
    h = jnp.maximum(h, 0.0)
    h = jnp.maximum(jnp.dot(h, w2_ref[...], preferred_element_type=jnp.float32) + b2_ref[...], 0.0)
    h = jnp.maximum(jnp.dot(h, w3_ref[...], preferred_element_type=jnp.float32) + b3_ref[...], 0.0)
    out_ref[...] = h


def _edge_mlp(tr, rel8, wrel, wdist, wunit, w2, b2, w3, b3):
    wb = lambda i: (0, 0)
    return pl.pallas_call(
        _edge_body,
        grid=(E // _BE,),
        in_specs=[
            pl.BlockSpec((_BE, H), lambda i: (i, 0)),
            pl.BlockSpec((_BE, 8), lambda i: (i, 0)),
            pl.BlockSpec((8, H), wb),
            pl.BlockSpec((1, H), wb),
            pl.BlockSpec((8, H), wb),
            pl.BlockSpec((H, H), wb),
            pl.BlockSpec((1, H), wb),
            pl.BlockSpec((H, H), wb),
            pl.BlockSpec((1, H), wb),
        ],
        out_specs=pl.BlockSpec((_BE, H), lambda i: (i, 0)),
        out_shape=jax.ShapeDtypeStruct((E, H), jnp.float32),
    )(tr, rel8, wrel, wdist, wunit, w2, b2, w3, b3)


def _scatter_body(msg_hbm, row_hbm, zero_hbm, out_hbm, idx_v, msg_v, agg_sh, sem):
    c = lax.axis_index("c")
    s = lax.axis_index("s")
    rbase = s * _RPT
    pltpu.sync_copy(zero_hbm.at[pl.ds(rbase, _RPT)], agg_sh.at[pl.ds(rbase, _RPT)])

    @pl.when(s == _NS - 1)
    def _init_tail():
        tl = pl.ds(_NS * _RPT, _RTAIL)
        pltpu.sync_copy(zero_hbm.at[tl], agg_sh.at[tl])

    plsc.subcore_barrier()

    ebase = c * (E // _NC) + s * _PW

    def body(k, carry):
        b = ebase + k * _GC
        pltpu.sync_copy(row_hbm.at[pl.ds(b, _GC)], idx_v)
        cp = pltpu.async_copy(msg_hbm.at[pl.ds(b, _GC)], msg_v, sem)
        cp.wait()
        pltpu.sync_copy(msg_v, agg_sh.at[idx_v], add=True)
        return carry

    lax.fori_loop(0, _PW // _GC, body, 0)
    plsc.subcore_barrier()
    pltpu.sync_copy(agg_sh.at[pl.ds(rbase, _RPT)],
                    out_hbm.at[pl.ds(c * N + rbase, _RPT)])

    @pl.when(s == _NS - 1)
    def _out_tail():
        pltpu.sync_copy(agg_sh.at[pl.ds(_NS * _RPT, _RTAIL)],
                        out_hbm.at[pl.ds(c * N + _NS * _RPT, _RTAIL)])


_scatter = pl.kernel(
    _scatter_body,
    out_type=jax.ShapeDtypeStruct((_NC * N, H), jnp.float32),
    mesh=plsc.VectorSubcoreMesh(core_axis_name="c", subcore_axis_name="s"),
    scratch_types=[
        pltpu.VMEM((_GC,), jnp.int32),
        pltpu.VMEM((_GC, H), jnp.float32),
        pltpu.VMEM_SHARED((N, H), jnp.float32),
        pltpu.SemaphoreType.DMA,
    ],
)


def _node_body(x_ref, a0_ref, a1_ref, w1a_ref, w1b_ref, b1_ref,
               w2_ref, b2_ref, w3_ref, b3_ref, w4_ref, b4_ref, out_ref):
    agg = a0_ref[...] + a1_ref[...]
    h = (jnp.dot(x_ref[...], w1a_ref[...], preferred_element_type=jnp.float32)
         + jnp.dot(agg, w1b_ref[...], preferred_element_type=jnp.float32)
         + b1_ref[...])
    h = jnp.maximum(h, 0.0)
    h = jnp.maximum(jnp.dot(h, w2_ref[...], preferred_element_type=jnp.float32) + b2_ref[...], 0.0)
    h = jnp.maximum(jnp.dot(h, w3_ref[...], preferred_element_type=jnp.float32) + b3_ref[...], 0.0)
    out_ref[...] = jnp.dot(h, w4_ref[...], preferred_element_type=jnp.float32) + b4_ref[...]


def _node_mlp(x, a0, a1, cw1a, cw1b, cb1, cw2, cb2, cw3, cb3, cw4, cb4):
    wb = lambda i: (0, 0)
    return pl.pallas_call(
        _node_body,
        grid=(N // _BN,),
        in_specs=[
            pl.BlockSpec((_BN, D), lambda i: (i, 0)),
            pl.BlockSpec((_BN, H), lambda i: (i, 0)),
            pl.BlockSpec((_BN, H), lambda i: (i, 0)),
            pl.BlockSpec((D, H), wb),
            pl.BlockSpec((H, H), wb),
            pl.BlockSpec((1, H), wb),
            pl.BlockSpec((H, H), wb),
            pl.BlockSpec((1, H), wb),
            pl.BlockSpec((H, H), wb),
            pl.BlockSpec((1, H), wb),
            pl.BlockSpec((H, D), wb),
            pl.BlockSpec((1, D), wb),
        ],
        out_specs=pl.BlockSpec((_BN, D), lambda i: (i, 0)),
        out_shape=jax.ShapeDtypeStruct((N, D), jnp.float32),
    )(x, a0, a1, cw1a, cw1b, cb1, cw2, cb2, cw3, cb3, cw4, cb4)


def kernel(x, edge_index, mw1, mb1, mw2, mb2, mw3, mb3,
           cw1, cb1, cw2, cb2, cw3, cb3, cw4, cb4):
    row = edge_index[0]
    col = edge_index[1]

    w1a = mw1[:D]
    w1b = mw1[D:]
    z5 = jnp.zeros((5, H), jnp.float32)
    wrel = jnp.concatenate([w1b[0:3], z5], axis=0)
    wdist = w1b[3:4]
    wunit = jnp.concatenate([w1b[4:7], z5], axis=0)

    t = _prep(x, w1a, mb1.reshape(1, H))
    tr, rel8 = _gather(t, x[:, 0], x[:, 1], x[:, 2], row, col)
    msg = _edge_mlp(tr, rel8, wrel, wdist, wunit,
                    mw2, mb2.reshape(1, H), mw3, mb3.reshape(1, H))
    zeros = jnp.zeros((N, H), jnp.float32)
    agg2 = _scatter(msg, row, zeros)
    return _node_mlp(x, agg2[:N], agg2[N:], cw1[:D], cw1[D:],
                     cb1.reshape(1, H), cw2, cb2.reshape(1, H),
                     cw3, cb3.reshape(1, H), cw4, cb4.reshape(1, D))


# speedup vs baseline: 4.2335x; 1.7045x over previous
import jax
import jax.numpy as jnp
from jax import lax
from jax.experimental import pallas as pl
from jax.experimental.pallas import tpu as pltpu
from jax.experimental.pallas import tpu_sc as plsc

N = 10000
E = 320000
D = 128
H = 128

_NC = 2
_NS = 16
_NW = _NC * _NS

_E2 = E // 2
_PW = _E2 // _NW
_CH = 128
_NF = _PW // _CH
_TOFF = _PW - 16

_RPT = 624
_RTAIL = N - _NS * _RPT

_BE = 2000
_BN = 1000


def _prep_body(x_ref, w_ref, b_ref, t_ref):
    xb = x_ref[...]
    t_ref[...] = jnp.dot(xb, w_ref[...], preferred_element_type=jnp.float32) + b_ref[...]


def _prep(x, w1a, mb1):
    return pl.pallas_call(
        _prep_body,
        grid=(N // _BN,),
        in_specs=[
            pl.BlockSpec((_BN, D), lambda i: (i, 0)),
            pl.BlockSpec((D, H), lambda i: (0, 0)),
            pl.BlockSpec((1, H), lambda i: (0, 0)),
        ],
        out_specs=pl.BlockSpec((_BN, H), lambda i: (i, 0)),
        out_shape=jax.ShapeDtypeStruct((N, H), jnp.float32),
    )(x, w1a, mb1)


def _gather_body(t_hbm, xtab_hbm, ytab_hbm, ztab_hbm, row_hbm, col_hbm,
                 tr_hbm, rel_hbm,
                 xtab_v, ytab_v, ztab_v, rows_v, cols_v,
                 tr0, tr1, rel0, rel1, sg0, sg1, sw0, sw1, sr0, sr1):
    wid = lax.axis_index("s") * _NC + lax.axis_index("c")
    base0 = wid * _PW
    c1 = pltpu.async_copy(xtab_hbm, xtab_v, sg0)
    c2 = pltpu.async_copy(ytab_hbm, ytab_v, sg1)
    c3 = pltpu.async_copy(ztab_hbm, ztab_v, sw0)
    c4 = pltpu.async_copy(row_hbm.at[pl.ds(base0, _PW)], rows_v, sw1)
    c5 = pltpu.async_copy(col_hbm.at[pl.ds(base0, _PW)], cols_v, sr0)
    c1.wait()
    c2.wait()
    c3.wait()
    c4.wait()
    c5.wait()

    def compute_rel(off, n16, rel_buf):
        for j in range(n16):
            ir = rows_v[pl.ds(off + j * 16, 16)]
            ic = cols_v[pl.ds(off + j * 16, 16)]
            erow = lax.iota(jnp.int32, 16) + j * 16
            for f, tab in ((0, xtab_v), (1, ytab_v), (2, ztab_v)):
                r = plsc.load_gather(tab, [ic]) - plsc.load_gather(tab, [ir])
                plsc.store_scatter(rel_buf, [erow, jnp.full((16,), f, jnp.int32)], r)

    def body(i, carry):
        o0 = (2 * i) * _CH
        o1 = o0 + _CH
        g0 = pltpu.async_copy(t_hbm.at[rows_v.at[pl.ds(o0, _CH)]], tr0, sg0)
        g1 = pltpu.async_copy(t_hbm.at[rows_v.at[pl.ds(o1, _CH)]], tr1, sg1)
        compute_rel(o0, _CH // 16, rel0)
        g0.wait()
        w0 = pltpu.async_copy(tr0, tr_hbm.at[pl.ds(base0 + o0, _CH)], sw0)
        r0 = pltpu.async_copy(rel0, rel_hbm.at[pl.ds(base0 + o0, _CH)], sr0)
        compute_rel(o1, _CH // 16, rel1)
        g1.wait()
        w1 = pltpu.async_copy(tr1, tr_hbm.at[pl.ds(base0 + o1, _CH)], sw1)
        r1 = pltpu.async_copy(rel1, rel_hbm.at[pl.ds(base0 + o1, _CH)], sr1)
        w0.wait()
        r0.wait()
        w1.wait()
        r1.wait()
        return carry

    lax.fori_loop(0, _NF // 2, body, 0)

    o0 = (_NF - 1) * _CH
    g0 = pltpu.async_copy(t_hbm.at[rows_v.at[pl.ds(o0, _CH)]], tr0, sg0)
    gt = pltpu.async_copy(t_hbm.at[rows_v.at[pl.ds(_TOFF, 16)]],
                          tr1.at[pl.ds(0, 16)], sg1)
    compute_rel(o0, _CH // 16, rel0)
    g0.wait()
    w0 = pltpu.async_copy(tr0, tr_hbm.at[pl.ds(base0 + o0, _CH)], sw0)
    r0 = pltpu.async_copy(rel0, rel_hbm.at[pl.ds(base0 + o0, _CH)], sr0)
    compute_rel(_TOFF, 1, rel1)
    gt.wait()
    w0.wait()
    r0.wait()
    pltpu.sync_copy(tr1.at[pl.ds(0, 16)], tr_hbm.at[pl.ds(base0 + _TOFF, 16)])
    pltpu.sync_copy(rel1.at[pl.ds(0, 16)], rel_hbm.at[pl.ds(base0 + _TOFF, 16)])


_gather = pl.kernel(
    _gather_body,
    out_type=(
        jax.ShapeDtypeStruct((_E2, H), jnp.float32),
        jax.ShapeDtypeStruct((_E2, 8), jnp.float32),
    ),
    mesh=plsc.VectorSubcoreMesh(core_axis_name="c", subcore_axis_name="s"),
    compiler_params=pltpu.CompilerParams(needs_layout_passes=False),
    scratch_types=[
        pltpu.VMEM((N,), jnp.float32),
        pltpu.VMEM((N,), jnp.float32),
        pltpu.VMEM((N,), jnp.float32),
        pltpu.VMEM((_PW,), jnp.int32),
        pltpu.VMEM((_PW,), jnp.int32),
        pltpu.VMEM((_CH, H), jnp.float32),
        pltpu.VMEM((_CH, H), jnp.float32),
        pltpu.VMEM((_CH, 8), jnp.float32),
        pltpu.VMEM((_CH, 8), jnp.float32),
        pltpu.SemaphoreType.DMA,
        pltpu.SemaphoreType.DMA,
        pltpu.SemaphoreType.DMA,
        pltpu.SemaphoreType.DMA,
        pltpu.SemaphoreType.DMA,
        pltpu.SemaphoreType.DMA,
    ],
)


def _edge_body(tr_ref, rel_ref, wrel_ref, wdist_ref, wunit_ref,
               w2_ref, b2_ref, w3_ref, b3_ref, out_ref):
    lane = lax.broadcasted_iota(jnp.int32, (_BE, 8), 1)
    rel = jnp.where(lane < 3, rel_ref[...], 0.0)
    d2 = jnp.sum(rel * rel, axis=1, keepdims=True)
    dist = jnp.sqrt(d2) + 1e-12
    unit = rel / dist
    h = (tr_ref[...]
         + jnp.dot(rel, wrel_ref[...], preferred_element_type=jnp.float32)
         + jnp.dot(unit, wunit_ref[...], preferred_element_type=jnp.float32)
         + dist * wdist_ref[...])
    h = jnp.maximum(h, 0.0)
    h = jnp.maximum(jnp.dot(h, w2_ref[...], preferred_element_type=jnp.float32) + b2_ref[...], 0.0)
    h = jnp.maximum(jnp.dot(h, w3_ref[...], preferred_element_type=jnp.float32) + b3_ref[...], 0.0)
    out_ref[...] = h


def _edge_mlp(tr, rel8, wrel, wdist, wunit, w2, b2, w3, b3):
    wb = lambda i: (0, 0)
    return pl.pallas_call(
        _edge_body,
        grid=(_E2 // _BE,),
        in_specs=[
            pl.BlockSpec((_BE, H), lambda i: (i, 0)),
            pl.BlockSpec((_BE, 8), lambda i: (i, 0)),
            pl.BlockSpec((8, H), wb),
            pl.BlockSpec((1, H), wb),
            pl.BlockSpec((8, H), wb),
            pl.BlockSpec((H, H), wb),
            pl.BlockSpec((1, H), wb),
            pl.BlockSpec((H, H), wb),
            pl.BlockSpec((1, H), wb),
        ],
        out_specs=pl.BlockSpec((_BE, H), lambda i: (i, 0)),
        out_shape=jax.ShapeDtypeStruct((_E2, H), jnp.float32),
    )(tr, rel8, wrel, wdist, wunit, w2, b2, w3, b3)


_TCH = _E2 // _CH
_WCH = _TCH // _NW
_XTRA = _TCH - _WCH * _NW
_WCHA = ((_WCH + 8 + 7) // 8) * 8
_R2D = ((_TCH + 7) // 8) * 8


def _scatter_body(msg_hbm, row2d_hbm, init_hbm, out_hbm,
                  idx2d_v, msg0, msg1, agg_sh, sm0, sm1):
    c = lax.axis_index("c")
    s = lax.axis_index("s")
    wid = s * _NC + c
    rbase = s * _RPT
    pltpu.sync_copy(init_hbm.at[pl.ds(c * N + rbase, _RPT)],
                    agg_sh.at[pl.ds(rbase, _RPT)])

    @pl.when(s == _NS - 1)
    def _init_tail():
        pltpu.sync_copy(init_hbm.at[pl.ds(c * N + _NS * _RPT, _RTAIL)],
                        agg_sh.at[pl.ds(_NS * _RPT, _RTAIL)])

    start = wid * _WCH
    astart = (start // 8) * 8
    doff = start - astart
    pltpu.sync_copy(row2d_hbm.at[pl.ds(astart, _WCHA)],
                    idx2d_v.at[pl.ds(0, _WCHA)])

    @pl.when(wid < _XTRA)
    def _tail_idx():
        pltpu.sync_copy(row2d_hbm.at[pl.ds(_NW * _WCH, 8)],
                        idx2d_v.at[pl.ds(_WCHA, 8)])

    plsc.subcore_barrier()

    def body(i, carry):
        j0 = 2 * i
        j1 = j0 + 1
        m0 = pltpu.async_copy(msg_hbm.at[pl.ds((start + j0) * _CH, _CH)], msg0, sm0)
        m1 = pltpu.async_copy(msg_hbm.at[pl.ds((start + j1) * _CH, _CH)], msg1, sm1)
        m0.wait()
        pltpu.sync_copy(msg0, agg_sh.at[idx2d_v.at[doff + j0]], add=True)
        m1.wait()
        pltpu.sync_copy(msg1, agg_sh.at[idx2d_v.at[doff + j1]], add=True)
        return carry

    lax.fori_loop(0, _WCH // 2, body, 0)

    jl = _WCH - 1
    ml = pltpu.async_copy(msg_hbm.at[pl.ds((start + jl) * _CH, _CH)], msg0, sm0)
    ml.wait()
    pltpu.sync_copy(msg0, agg_sh.at[idx2d_v.at[doff + jl]], add=True)

    @pl.when(wid < _XTRA)
    def _tail_scatter():
        b = (_NW * _WCH + wid) * _CH
        mt = pltpu.async_copy(msg_hbm.at[pl.ds(b, _CH)], msg1, sm1)
        mt.wait()
        pltpu.sync_copy(msg1, agg_sh.at[idx2d_v.at[_WCHA + wid]], add=True)

    plsc.subcore_barrier()
    pltpu.sync_copy(agg_sh.at[pl.ds(rbase, _RPT)],
                    out_hbm.at[pl.ds(c * N + rbase, _RPT)])

    @pl.when(s == _NS - 1)
    def _out_tail():
        pltpu.sync_copy(agg_sh.at[pl.ds(_NS * _RPT, _RTAIL)],
                        out_hbm.at[pl.ds(c * N + _NS * _RPT, _RTAIL)])


_scatter = pl.kernel(
    _scatter_body,
    out_type=jax.ShapeDtypeStruct((_NC * N, H), jnp.float32),
    mesh=plsc.VectorSubcoreMesh(core_axis_name="c", subcore_axis_name="s"),
    scratch_types=[
        pltpu.VMEM((_WCHA + 8, _CH), jnp.int32),
        pltpu.VMEM((_CH, H), jnp.float32),
        pltpu.VMEM((_CH, H), jnp.float32),
        pltpu.VMEM_SHARED((N, H), jnp.float32),
        pltpu.SemaphoreType.DMA,
        pltpu.SemaphoreType.DMA,
    ],
)


def _node_body(x_ref, a0_ref, a1_ref, w1a_ref, w1b_ref, b1_ref,
               w2_ref, b2_ref, w3_ref, b3_ref, w4_ref, b4_ref, out_ref):
    agg = a0_ref[...] + a1_ref[...]
    h = (jnp.dot(x_ref[...], w1a_ref[...], preferred_element_type=jnp.float32)
         + jnp.dot(agg, w1b_ref[...], preferred_element_type=jnp.float32)
         + b1_ref[...])
    h = jnp.maximum(h, 0.0)
    h = jnp.maximum(jnp.dot(h, w2_ref[...], preferred_element_type=jnp.float32) + b2_ref[...], 0.0)
    h = jnp.maximum(jnp.dot(h, w3_ref[...], preferred_element_type=jnp.float32) + b3_ref[...], 0.0)
    out_ref[...] = jnp.dot(h, w4_ref[...], preferred_element_type=jnp.float32) + b4_ref[...]


def _node_mlp(x, a0, a1, cw1a, cw1b, cb1, cw2, cb2, cw3, cb3, cw4, cb4):
    wb = lambda i: (0, 0)
    return pl.pallas_call(
        _node_body,
        grid=(N // _BN,),
        in_specs=[
            pl.BlockSpec((_BN, D), lambda i: (i, 0)),
            pl.BlockSpec((_BN, H), lambda i: (i, 0)),
            pl.BlockSpec((_BN, H), lambda i: (i, 0)),
            pl.BlockSpec((D, H), wb),
            pl.BlockSpec((H, H), wb),
            pl.BlockSpec((1, H), wb),
            pl.BlockSpec((H, H), wb),
            pl.BlockSpec((1, H), wb),
            pl.BlockSpec((H, H), wb),
            pl.BlockSpec((1, H), wb),
            pl.BlockSpec((H, D), wb),
            pl.BlockSpec((1, D), wb),
        ],
        out_specs=pl.BlockSpec((_BN, D), lambda i: (i, 0)),
        out_shape=jax.ShapeDtypeStruct((N, D), jnp.float32),
    )(x, a0, a1, cw1a, cw1b, cb1, cw2, cb2, cw3, cb3, cw4, cb4)


def kernel(x, edge_index, mw1, mb1, mw2, mb2, mw3, mb3,
           cw1, cb1, cw2, cb2, cw3, cb3, cw4, cb4):
    row = edge_index[0]
    col = edge_index[1]

    w1a = mw1[:D]
    w1b = mw1[D:]
    z5 = jnp.zeros((5, H), jnp.float32)
    wrel = jnp.concatenate([w1b[0:3], z5], axis=0)
    wdist = w1b[3:4]
    wunit = jnp.concatenate([w1b[4:7], z5], axis=0)

    t = _prep(x, w1a, mb1.reshape(1, H))
    xs, ys, zs = x[:, 0], x[:, 1], x[:, 2]
    mw2b = mb2.reshape(1, H)
    mw3b = mb3.reshape(1, H)

    halves = []
    for h in range(2):
        rh = lax.slice(row, (h * _E2,), ((h + 1) * _E2,))
        ch = lax.slice(col, (h * _E2,), ((h + 1) * _E2,))
        tr, rel8 = _gather(t, xs, ys, zs, rh, ch)
        msg = _edge_mlp(tr, rel8, wrel, wdist, wunit, mw2, mw2b, mw3, mw3b)
        r2d = jnp.pad(rh.reshape(_TCH, _CH), ((0, _R2D - _TCH), (0, 0)))
        halves.append((msg, r2d))

    acc = jnp.zeros((_NC * N, H), jnp.float32)
    for msg, r2d in halves:
        acc = _scatter(msg, r2d, acc)

    return _node_mlp(x, acc[:N], acc[N:], cw1[:D], cw1[D:],
                     cb1.reshape(1, H), cw2, cb2.reshape(1, H),
                     cw3, cb3.reshape(1, H), cw4, cb4.reshape(1, D))


# speedup vs baseline: 4.2482x; 1.0035x over previous
import jax
import jax.numpy as jnp
from jax import lax
from jax.experimental import pallas as pl
from jax.experimental.pallas import tpu as pltpu
from jax.experimental.pallas import tpu_sc as plsc

N = 10000
E = 320000
D = 128
H = 128

_NC = 2
_NS = 16
_NW = _NC * _NS

_CH = 128
_PARTS = (64000, 128000, 128000)

_RPT = 624
_RTAIL = N - _NS * _RPT

_BE = 2000
_BN = 1000


def _prep_body(x_ref, w_ref, b_ref, t_ref):
    xb = x_ref[...]
    t_ref[...] = jnp.dot(xb, w_ref[...], preferred_element_type=jnp.float32) + b_ref[...]


def _prep(x, w1a, mb1):
    return pl.pallas_call(
        _prep_body,
        grid=(N // _BN,),
        in_specs=[
            pl.BlockSpec((_BN, D), lambda i: (i, 0)),
            pl.BlockSpec((D, H), lambda i: (0, 0)),
            pl.BlockSpec((1, H), lambda i: (0, 0)),
        ],
        out_specs=pl.BlockSpec((_BN, H), lambda i: (i, 0)),
        out_shape=jax.ShapeDtypeStruct((N, H), jnp.float32),
    )(x, w1a, mb1)


def _make_gather(nchunks):
    wch = nchunks // _NW
    xtra = nchunks - wch * _NW
    pw = wch * _CH
    ne = nchunks * _CH

    def body_fn(t_hbm, xtab_hbm, ytab_hbm, ztab_hbm, row_hbm, col_hbm,
                tr_hbm, rel_hbm,
                xtab_v, ytab_v, ztab_v, rows_v, cols_v,
                tr0, tr1, rel0, rel1, sg0, sg1, sw0, sw1, sr0, sr1):
        wid = lax.axis_index("s") * _NC + lax.axis_index("c")
        base0 = wid * pw
        ebase = (_NW * wch + wid) * _CH
        c1 = pltpu.async_copy(xtab_hbm, xtab_v, sg0)
        c2 = pltpu.async_copy(ytab_hbm, ytab_v, sg1)
        c3 = pltpu.async_copy(ztab_hbm, ztab_v, sw0)
        c4 = pltpu.async_copy(row_hbm.at[pl.ds(base0, pw)],
                              rows_v.at[pl.ds(0, pw)], sw1)
        c5 = pltpu.async_copy(col_hbm.at[pl.ds(base0, pw)],
                              cols_v.at[pl.ds(0, pw)], sr0)

        @pl.when(wid < xtra)
        def _stage_extra():
            pltpu.sync_copy(row_hbm.at[pl.ds(ebase, _CH)],
                            rows_v.at[pl.ds(pw, _CH)])
            pltpu.sync_copy(col_hbm.at[pl.ds(ebase, _CH)],
                            cols_v.at[pl.ds(pw, _CH)])

        c1.wait()
        c2.wait()
        c3.wait()
        c4.wait()
        c5.wait()

        def compute_rel(off, rel_buf):
            for j in range(_CH // 16):
                ir = rows_v[pl.ds(off + j * 16, 16)]
                ic = cols_v[pl.ds(off + j * 16, 16)]
                erow = lax.iota(jnp.int32, 16) + j * 16
                for f, tab in ((0, xtab_v), (1, ytab_v), (2, ztab_v)):
                    r = plsc.load_gather(tab, [ic]) - plsc.load_gather(tab, [ir])
                    plsc.store_scatter(rel_buf, [erow, jnp.full((16,), f, jnp.int32)], r)

        def body(i, carry):
            o0 = (2 * i) * _CH
            o1 = o0 + _CH
            g0 = pltpu.async_copy(t_hbm.at[rows_v.at[pl.ds(o0, _CH)]], tr0, sg0)
            g1 = pltpu.async_copy(t_hbm.at[rows_v.at[pl.ds(o1, _CH)]], tr1, sg1)
            compute_rel(o0, rel0)
            g0.wait()
            w0 = pltpu.async_copy(tr0, tr_hbm.at[pl.ds(base0 + o0, _CH)], sw0)
            r0 = pltpu.async_copy(rel0, rel_hbm.at[pl.ds(base0 + o0, _CH)], sr0)
            compute_rel(o1, rel1)
            g1.wait()
            w1 = pltpu.async_copy(tr1, tr_hbm.at[pl.ds(base0 + o1, _CH)], sw1)
            r1 = pltpu.async_copy(rel1, rel_hbm.at[pl.ds(base0 + o1, _CH)], sr1)
            w0.wait()
            r0.wait()
            w1.wait()
            r1.wait()
            return carry

        lax.fori_loop(0, wch // 2, body, 0)

        if wch % 2:
            o0 = (wch - 1) * _CH
            g0 = pltpu.async_copy(t_hbm.at[rows_v.at[pl.ds(o0, _CH)]], tr0, sg0)
            compute_rel(o0, rel0)
            g0.wait()
            w0 = pltpu.async_copy(tr0, tr_hbm.at[pl.ds(base0 + o0, _CH)], sw0)
            r0 = pltpu.async_copy(rel0, rel_hbm.at[pl.ds(base0 + o0, _CH)], sr0)
            w0.wait()
            r0.wait()

        @pl.when(wid < xtra)
        def _extra_chunk():
            g1 = pltpu.async_copy(t_hbm.at[rows_v.at[pl.ds(pw, _CH)]], tr1, sg1)
            compute_rel(pw, rel1)
            g1.wait()
            pltpu.sync_copy(tr1, tr_hbm.at[pl.ds(ebase, _CH)])
            pltpu.sync_copy(rel1, rel_hbm.at[pl.ds(ebase, _CH)])

    return pl.kernel(
        body_fn,
        out_type=(
            jax.ShapeDtypeStruct((ne, H), jnp.float32),
            jax.ShapeDtypeStruct((ne, 8), jnp.float32),
        ),
        mesh=plsc.VectorSubcoreMesh(core_axis_name="c", subcore_axis_name="s"),
        compiler_params=pltpu.CompilerParams(needs_layout_passes=False),
        scratch_types=[
            pltpu.VMEM((N,), jnp.float32),
            pltpu.VMEM((N,), jnp.float32),
            pltpu.VMEM((N,), jnp.float32),
            pltpu.VMEM((pw + _CH,), jnp.int32),
            pltpu.VMEM((pw + _CH,), jnp.int32),
            pltpu.VMEM((_CH, H), jnp.float32),
            pltpu.VMEM((_CH, H), jnp.float32),
            pltpu.VMEM((_CH, 8), jnp.float32),
            pltpu.VMEM((_CH, 8), jnp.float32),
            pltpu.SemaphoreType.DMA,
            pltpu.SemaphoreType.DMA,
            pltpu.SemaphoreType.DMA,
            pltpu.SemaphoreType.DMA,
            pltpu.SemaphoreType.DMA,
            pltpu.SemaphoreType.DMA,
        ],
    )


_gathers = {ne: _make_gather(ne // _CH) for ne in set(_PARTS)}


def _edge_body(tr_ref, rel_ref, wrel_ref, wdist_ref, wunit_ref,
               w2_ref, b2_ref, w3_ref, b3_ref, out_ref):
    lane = lax.broadcasted_iota(jnp.int32, (_BE, 8), 1)
    rel = jnp.where(lane < 3, rel_ref[...], 0.0)
    d2 = jnp.sum(rel * rel, axis=1, keepdims=True)
    dist = jnp.sqrt(d2) + 1e-12
    unit = rel / dist
    h = (tr_ref[...]
         + jnp.dot(rel, wrel_ref[...], preferred_element_type=jnp.float32)
         + jnp.dot(unit, wunit_ref[...], preferred_element_type=jnp.float32)
         + dist * wdist_ref[...])
    h = jnp.maximum(h, 0.0)
    h = jnp.maximum(jnp.dot(h, w2_ref[...], preferred_element_type=jnp.float32) + b2_ref[...], 0.0)
    h = jnp.maximum(jnp.dot(h, w3_ref[...], preferred_element_type=jnp.float32) + b3_ref[...], 0.0)
    out_ref[...] = h


def _edge_mlp(tr, rel8, wrel, wdist, wunit, w2, b2, w3, b3):
    ne = tr.shape[0]
    wb = lambda i: (0, 0)
    return pl.pallas_call(
        _edge_body,
        grid=(ne // _BE,),
        in_specs=[
            pl.BlockSpec((_BE, H), lambda i: (i, 0)),
            pl.BlockSpec((_BE, 8), lambda i: (i, 0)),
            pl.BlockSpec((8, H), wb),
            pl.BlockSpec((1, H), wb),
            pl.BlockSpec((8, H), wb),
            pl.BlockSpec((H, H), wb),
            pl.BlockSpec((1, H), wb),
            pl.BlockSpec((H, H), wb),
            pl.BlockSpec((1, H), wb),
        ],
        out_specs=pl.BlockSpec((_BE, H), lambda i: (i, 0)),
        out_shape=jax.ShapeDtypeStruct((ne, H), jnp.float32),
    )(tr, rel8, wrel, wdist, wunit, w2, b2, w3, b3)


def _make_scatter(nchunks):
    wch = nchunks // _NW
    xtra = nchunks - wch * _NW
    wcha = ((wch + 8 + 7) // 8) * 8
    xbase = (_NW * wch // 8) * 8
    exdoff = _NW * wch - xbase
    xw = ((exdoff + xtra + 7) // 8) * 8 if xtra else 0
    r2d = max(((nchunks + 7) // 8) * 8, xbase + xw)

    def body_fn(msg_hbm, row2d_hbm, init_hbm, out_hbm,
                idx2d_v, msg0, msg1, agg_sh, sm0, sm1):
        c = lax.axis_index("c")
        s = lax.axis_index("s")
        wid = s * _NC + c
        rbase = s * _RPT
        pltpu.sync_copy(init_hbm.at[pl.ds(c * N + rbase, _RPT)],
                        agg_sh.at[pl.ds(rbase, _RPT)])

        @pl.when(s == _NS - 1)
        def _init_tail():
            pltpu.sync_copy(init_hbm.at[pl.ds(c * N + _NS * _RPT, _RTAIL)],
                            agg_sh.at[pl.ds(_NS * _RPT, _RTAIL)])

        start = wid * wch
        astart = (start // 8) * 8
        doff = start - astart
        pltpu.sync_copy(row2d_hbm.at[pl.ds(astart, wcha)],
                        idx2d_v.at[pl.ds(0, wcha)])

        @pl.when(wid < xtra)
        def _xtra_idx():
            pltpu.sync_copy(row2d_hbm.at[pl.ds(xbase, xw)],
                            idx2d_v.at[pl.ds(wcha, xw)])

        plsc.subcore_barrier()

        def body(i, carry):
            j0 = 2 * i
            j1 = j0 + 1
            m0 = pltpu.async_copy(msg_hbm.at[pl.ds((start + j0) * _CH, _CH)], msg0, sm0)
            m1 = pltpu.async_copy(msg_hbm.at[pl.ds((start + j1) * _CH, _CH)], msg1, sm1)
            m0.wait()
            pltpu.sync_copy(msg0, agg_sh.at[idx2d_v.at[doff + j0]], add=True)
            m1.wait()
            pltpu.sync_copy(msg1, agg_sh.at[idx2d_v.at[doff + j1]], add=True)
            return carry

        lax.fori_loop(0, wch // 2, body, 0)

        if wch % 2:
            jl = wch - 1
            ml = pltpu.async_copy(msg_hbm.at[pl.ds((start + jl) * _CH, _CH)], msg0, sm0)
            ml.wait()
            pltpu.sync_copy(msg0, agg_sh.at[idx2d_v.at[doff + jl]], add=True)

        @pl.when(wid < xtra)
        def _xtra_scatter():
            b = (_NW * wch + wid) * _CH
            mt = pltpu.async_copy(msg_hbm.at[pl.ds(b, _CH)], msg1, sm1)
            mt.wait()
            pltpu.sync_copy(msg1, agg_sh.at[idx2d_v.at[wcha + exdoff + wid]], add=True)

        plsc.subcore_barrier()
        pltpu.sync_copy(agg_sh.at[pl.ds(rbase, _RPT)],
                        out_hbm.at[pl.ds(c * N + rbase, _RPT)])

        @pl.when(s == _NS - 1)
        def _out_tail():
            pltpu.sync_copy(agg_sh.at[pl.ds(_NS * _RPT, _RTAIL)],
                            out_hbm.at[pl.ds(c * N + _NS * _RPT, _RTAIL)])

    kern = pl.kernel(
        body_fn,
        out_type=jax.ShapeDtypeStruct((_NC * N, H), jnp.float32),
        mesh=plsc.VectorSubcoreMesh(core_axis_name="c", subcore_axis_name="s"),
        scratch_types=[
            pltpu.VMEM((wcha + max(xw, 8), _CH), jnp.int32),
            pltpu.VMEM((_CH, H), jnp.float32),
            pltpu.VMEM((_CH, H), jnp.float32),
            pltpu.VMEM_SHARED((N, H), jnp.float32),
            pltpu.SemaphoreType.DMA,
            pltpu.SemaphoreType.DMA,
        ],
    )
    return kern, r2d


_scatters = {ne: _make_scatter(ne // _CH) for ne in set(_PARTS)}


def _node_body(x_ref, a0_ref, a1_ref, w1a_ref, w1b_ref, b1_ref,
               w2_ref, b2_ref, w3_ref, b3_ref, w4_ref, b4_ref, out_ref):
    agg = a0_ref[...] + a1_ref[...]
    h = (jnp.dot(x_ref[...], w1a_ref[...], preferred_element_type=jnp.float32)
         + jnp.dot(agg, w1b_ref[...], preferred_element_type=jnp.float32)
         + b1_ref[...])
    h = jnp.maximum(h, 0.0)
    h = jnp.maximum(jnp.dot(h, w2_ref[...], preferred_element_type=jnp.float32) + b2_ref[...], 0.0)
    h = jnp.maximum(jnp.dot(h, w3_ref[...], preferred_element_type=jnp.float32) + b3_ref[...], 0.0)
    out_ref[...] = jnp.dot(h, w4_ref[...], preferred_element_type=jnp.float32) + b4_ref[...]


def _node_mlp(x, a0, a1, cw1a, cw1b, cb1, cw2, cb2, cw3, cb3, cw4, cb4):
    wb = lambda i: (0, 0)
    return pl.pallas_call(
        _node_body,
        grid=(N // _BN,),
        in_specs=[
            pl.BlockSpec((_BN, D), lambda i: (i, 0)),
            pl.BlockSpec((_BN, H), lambda i: (i, 0)),
            pl.BlockSpec((_BN, H), lambda i: (i, 0)),
            pl.BlockSpec((D, H), wb),
            pl.BlockSpec((H, H), wb),
            pl.BlockSpec((1, H), wb),
            pl.BlockSpec((H, H), wb),
            pl.BlockSpec((1, H), wb),
            pl.BlockSpec((H, H), wb),
            pl.BlockSpec((1, H), wb),
            pl.BlockSpec((H, D), wb),
            pl.BlockSpec((1, D), wb),
        ],
        out_specs=pl.BlockSpec((_BN, D), lambda i: (i, 0)),
        out_shape=jax.ShapeDtypeStruct((N, D), jnp.float32),
    )(x, a0, a1, cw1a, cw1b, cb1, cw2, cb2, cw3, cb3, cw4, cb4)


def kernel(x, edge_index, mw1, mb1, mw2, mb2, mw3, mb3,
           cw1, cb1, cw2, cb2, cw3, cb3, cw4, cb4):
    row = edge_index[0]
    col = edge_index[1]

    w1a = mw1[:D]
    w1b = mw1[D:]
    z5 = jnp.zeros((5, H), jnp.float32)
    wrel = jnp.concatenate([w1b[0:3], z5], axis=0)
    wdist = w1b[3:4]
    wunit = jnp.concatenate([w1b[4:7], z5], axis=0)

    t = _prep(x, w1a, mb1.reshape(1, H))
    xs, ys, zs = x[:, 0], x[:, 1], x[:, 2]
    mw2b = mb2.reshape(1, H)
    mw3b = mb3.reshape(1, H)

    parts = []
    off = 0
    for ne in _PARTS:
        rh = lax.slice(row, (off,), (off + ne,))
        ch = lax.slice(col, (off,), (off + ne,))
        off += ne
        tr, rel8 = _gathers[ne](t, xs, ys, zs, rh, ch)
        msg = _edge_mlp(tr, rel8, wrel, wdist, wunit, mw2, mw2b, mw3, mw3b)
        nch = ne // _CH
        r2d_rows = _scatters[ne][1]
        r2d = jnp.pad(rh.reshape(nch, _CH), ((0, r2d_rows - nch), (0, 0)))
        parts.append((ne, msg, r2d))

    acc = jnp.zeros((_NC * N, H), jnp.float32)
    for ne, msg, r2d in parts:
        acc = _scatters[ne][0](msg, r2d, acc)

    return _node_mlp(x, acc[:N], acc[N:], cw1[:D], cw1[D:],
                     cb1.reshape(1, H), cw2, cb2.reshape(1, H),
                     cw3, cb3.reshape(1, H), cw4, cb4.reshape(1, D))


# speedup vs baseline: 4.4821x; 1.0550x over previous
import jax
import jax.numpy as jnp
from jax import lax
from jax.experimental import pallas as pl
from jax.experimental.pallas import tpu as pltpu
from jax.experimental.pallas import tpu_sc as plsc

N = 10000
E = 320000
D = 128
H = 128

_NC = 2
_NS = 16
_NW = _NC * _NS

_CH = 128
_PARTS = (64000, 128000, 128000)

_RPT = 624
_RTAIL = N - _NS * _RPT

_BE = 2000
_BN = 1000


def _prep_body(x_ref, w_ref, b_ref, t_ref):
    xb = x_ref[...]
    t_ref[...] = jnp.dot(xb, w_ref[...], preferred_element_type=jnp.float32) + b_ref[...]


def _prep(x, w1a, mb1):
    return pl.pallas_call(
        _prep_body,
        grid=(N // _BN,),
        in_specs=[
            pl.BlockSpec((_BN, D), lambda i: (i, 0)),
            pl.BlockSpec((D, H), lambda i: (0, 0)),
            pl.BlockSpec((1, H), lambda i: (0, 0)),
        ],
        out_specs=pl.BlockSpec((_BN, H), lambda i: (i, 0)),
        out_shape=jax.ShapeDtypeStruct((N, H), jnp.float32),
    )(x, w1a, mb1)


def _make_rel():
    nchunks = E // _CH
    wch = nchunks // _NW
    xtra = nchunks - wch * _NW
    pw = wch * _CH

    def body_fn(xtab_hbm, ytab_hbm, ztab_hbm, row_hbm, col_hbm, rel_hbm,
                xtab_v, ytab_v, ztab_v, rows_v, cols_v,
                rel0, rel1, sr0, sr1, si0, si1):
        wid = lax.axis_index("s") * _NC + lax.axis_index("c")
        base0 = wid * pw
        ebase = (_NW * wch + wid) * _CH
        c1 = pltpu.async_copy(xtab_hbm, xtab_v, sr0)
        c2 = pltpu.async_copy(ytab_hbm, ytab_v, sr1)
        c3 = pltpu.async_copy(ztab_hbm, ztab_v, si0)
        c4 = pltpu.async_copy(row_hbm.at[pl.ds(base0, pw)],
                              rows_v.at[pl.ds(0, pw)], si1)
        c5 = pltpu.async_copy(col_hbm.at[pl.ds(base0, pw)],
                              cols_v.at[pl.ds(0, pw)], sr0)

        @pl.when(wid < xtra)
        def _stage_extra():
            pltpu.sync_copy(row_hbm.at[pl.ds(ebase, _CH)],
                            rows_v.at[pl.ds(pw, _CH)])
            pltpu.sync_copy(col_hbm.at[pl.ds(ebase, _CH)],
                            cols_v.at[pl.ds(pw, _CH)])

        c1.wait()
        c2.wait()
        c3.wait()
        c4.wait()
        c5.wait()

        def compute_rel(off, rel_buf):
            for j in range(_CH // 16):
                ir = rows_v[pl.ds(off + j * 16, 16)]
                ic = cols_v[pl.ds(off + j * 16, 16)]
                erow = lax.iota(jnp.int32, 16) + j * 16
                for f, tab in ((0, xtab_v), (1, ytab_v), (2, ztab_v)):
                    r = plsc.load_gather(tab, [ic]) - plsc.load_gather(tab, [ir])
                    plsc.store_scatter(rel_buf, [erow, jnp.full((16,), f, jnp.int32)], r)

        def body(i, carry):
            o0 = (2 * i) * _CH
            o1 = o0 + _CH
            compute_rel(o0, rel0)
            r0 = pltpu.async_copy(rel0, rel_hbm.at[pl.ds(base0 + o0, _CH)], sr0)
            compute_rel(o1, rel1)
            r1 = pltpu.async_copy(rel1, rel_hbm.at[pl.ds(base0 + o1, _CH)], sr1)
            r0.wait()
            r1.wait()
            return carry

        lax.fori_loop(0, wch // 2, body, 0)

        if wch % 2:
            o0 = (wch - 1) * _CH
            compute_rel(o0, rel0)
            pltpu.sync_copy(rel0, rel_hbm.at[pl.ds(base0 + o0, _CH)])

        @pl.when(wid < xtra)
        def _extra_chunk():
            compute_rel(pw, rel1)
            pltpu.sync_copy(rel1, rel_hbm.at[pl.ds(ebase, _CH)])

    return pl.kernel(
        body_fn,
        out_type=jax.ShapeDtypeStruct((E, 8), jnp.float32),
        mesh=plsc.VectorSubcoreMesh(core_axis_name="c", subcore_axis_name="s"),
        compiler_params=pltpu.CompilerParams(needs_layout_passes=False),
        scratch_types=[
            pltpu.VMEM((N,), jnp.float32),
            pltpu.VMEM((N,), jnp.float32),
            pltpu.VMEM((N,), jnp.float32),
            pltpu.VMEM((pw + _CH,), jnp.int32),
            pltpu.VMEM((pw + _CH,), jnp.int32),
            pltpu.VMEM((_CH, 8), jnp.float32),
            pltpu.VMEM((_CH, 8), jnp.float32),
            pltpu.SemaphoreType.DMA,
            pltpu.SemaphoreType.DMA,
            pltpu.SemaphoreType.DMA,
            pltpu.SemaphoreType.DMA,
        ],
    )


_rel = _make_rel()


def _make_gather(nchunks):
    wch = nchunks // _NW
    xtra = nchunks - wch * _NW
    pw = wch * _CH
    ne = nchunks * _CH

    def body_fn(t_hbm, row_hbm, tr_hbm,
                rows_v, tr0, tr1, t_sh, sg0, sg1, sw0, sw1):
        wid = lax.axis_index("s") * _NC + lax.axis_index("c")
        base0 = wid * pw
        ebase = (_NW * wch + wid) * _CH
        c4 = pltpu.async_copy(row_hbm.at[pl.ds(base0, pw)],
                              rows_v.at[pl.ds(0, pw)], sw1)

        @pl.when(wid < xtra)
        def _stage_extra():
            pltpu.sync_copy(row_hbm.at[pl.ds(ebase, _CH)],
                            rows_v.at[pl.ds(pw, _CH)])

        rb = lax.axis_index("s") * _RPT
        pltpu.sync_copy(t_hbm.at[pl.ds(rb, _RPT)], t_sh.at[pl.ds(rb, _RPT)])

        @pl.when(lax.axis_index("s") == _NS - 1)
        def _fill_tail():
            tl = _NS * _RPT
            pltpu.sync_copy(t_hbm.at[pl.ds(tl, _RTAIL)], t_sh.at[pl.ds(tl, _RTAIL)])

        plsc.subcore_barrier()
        c4.wait()

        def body(i, carry):
            o0 = (2 * i) * _CH
            o1 = o0 + _CH
            g0 = pltpu.async_copy(t_sh.at[rows_v.at[pl.ds(o0, _CH)]], tr0, sg0)
            g1 = pltpu.async_copy(t_sh.at[rows_v.at[pl.ds(o1, _CH)]], tr1, sg1)
            g0.wait()
            w0 = pltpu.async_copy(tr0, tr_hbm.at[pl.ds(base0 + o0, _CH)], sw0)
            g1.wait()
            w1 = pltpu.async_copy(tr1, tr_hbm.at[pl.ds(base0 + o1, _CH)], sw1)
            w0.wait()
            w1.wait()
            return carry

        lax.fori_loop(0, wch // 2, body, 0)

        if wch % 2:
            o0 = (wch - 1) * _CH
            g0 = pltpu.async_copy(t_sh.at[rows_v.at[pl.ds(o0, _CH)]], tr0, sg0)
            g0.wait()
            pltpu.sync_copy(tr0, tr_hbm.at[pl.ds(base0 + o0, _CH)])

        @pl.when(wid < xtra)
        def _extra_chunk():
            g1 = pltpu.async_copy(t_sh.at[rows_v.at[pl.ds(pw, _CH)]], tr1, sg1)
            g1.wait()
            pltpu.sync_copy(tr1, tr_hbm.at[pl.ds(ebase, _CH)])

    return pl.kernel(
        body_fn,
        out_type=jax.ShapeDtypeStruct((ne, H), jnp.float32),
        mesh=plsc.VectorSubcoreMesh(core_axis_name="c", subcore_axis_name="s"),
        compiler_params=pltpu.CompilerParams(needs_layout_passes=False),
        scratch_types=[
            pltpu.VMEM((pw + _CH,), jnp.int32),
            pltpu.VMEM((_CH, H), jnp.float32),
            pltpu.VMEM((_CH, H), jnp.float32),
            pltpu.VMEM_SHARED((N, H), jnp.float32),
            pltpu.SemaphoreType.DMA,
            pltpu.SemaphoreType.DMA,
            pltpu.SemaphoreType.DMA,
            pltpu.SemaphoreType.DMA,
        ],
    )


_gathers = {ne: _make_gather(ne // _CH) for ne in set(_PARTS)}


def _edge_body(tr_ref, rel_ref, wrel_ref, wdist_ref, wunit_ref,
               w2_ref, b2_ref, w3_ref, b3_ref, out_ref):
    lane = lax.broadcasted_iota(jnp.int32, (_BE, 8), 1)
    rel = jnp.where(lane < 3, rel_ref[...], 0.0)
    d2 = jnp.sum(rel * rel, axis=1, keepdims=True)
    dist = jnp.sqrt(d2) + 1e-12
    unit = rel / dist
    h = (tr_ref[...].astype(jnp.float32)
         + jnp.dot(rel, wrel_ref[...], preferred_element_type=jnp.float32)
         + jnp.dot(unit, wunit_ref[...], preferred_element_type=jnp.float32)
         + dist * wdist_ref[...])
    h = jnp.maximum(h, 0.0)
    h = jnp.maximum(jnp.dot(h, w2_ref[...], preferred_element_type=jnp.float32) + b2_ref[...], 0.0)
    h = jnp.maximum(jnp.dot(h, w3_ref[...], preferred_element_type=jnp.float32) + b3_ref[...], 0.0)
    out_ref[...] = h


def _edge_mlp(tr, rel8, ob, wrel, wdist, wunit, w2, b2, w3, b3):
    ne = tr.shape[0]
    wb = lambda i: (0, 0)
    return pl.pallas_call(
        _edge_body,
        grid=(ne // _BE,),
        in_specs=[
            pl.BlockSpec((_BE, H), lambda i: (i, 0)),
            pl.BlockSpec((_BE, 8), lambda i, _ob=ob: (i + _ob, 0)),
            pl.BlockSpec((8, H), wb),
            pl.BlockSpec((1, H), wb),
            pl.BlockSpec((8, H), wb),
            pl.BlockSpec((H, H), wb),
            pl.BlockSpec((1, H), wb),
            pl.BlockSpec((H, H), wb),
            pl.BlockSpec((1, H), wb),
        ],
        out_specs=pl.BlockSpec((_BE, H), lambda i: (i, 0)),
        out_shape=jax.ShapeDtypeStruct((ne, H), jnp.float32),
    )(tr, rel8, wrel, wdist, wunit, w2, b2, w3, b3)


def _make_scatter(nchunks):
    wch = nchunks // _NW
    xtra = nchunks - wch * _NW
    wcha = ((wch + 8 + 7) // 8) * 8
    xbase = (_NW * wch // 8) * 8
    exdoff = _NW * wch - xbase
    xw = ((exdoff + xtra + 7) // 8) * 8 if xtra else 0
    r2d = max(((nchunks + 7) // 8) * 8, xbase + xw)

    def body_fn(msg_hbm, row2d_hbm, init_hbm, out_hbm,
                idx2d_v, msg0, msg1, agg_sh, sm0, sm1):
        c = lax.axis_index("c")
        s = lax.axis_index("s")
        wid = s * _NC + c
        rbase = s * _RPT
        pltpu.sync_copy(init_hbm.at[pl.ds(c * N + rbase, _RPT)],
                        agg_sh.at[pl.ds(rbase, _RPT)])

        @pl.when(s == _NS - 1)
        def _init_tail():
            pltpu.sync_copy(init_hbm.at[pl.ds(c * N + _NS * _RPT, _RTAIL)],
                            agg_sh.at[pl.ds(_NS * _RPT, _RTAIL)])

        start = wid * wch
        astart = (start // 8) * 8
        doff = start - astart
        pltpu.sync_copy(row2d_hbm.at[pl.ds(astart, wcha)],
                        idx2d_v.at[pl.ds(0, wcha)])

        @pl.when(wid < xtra)
        def _xtra_idx():
            pltpu.sync_copy(row2d_hbm.at[pl.ds(xbase, xw)],
                            idx2d_v.at[pl.ds(wcha, xw)])

        plsc.subcore_barrier()

        def body(i, carry):
            j0 = 2 * i
            j1 = j0 + 1
            m0 = pltpu.async_copy(msg_hbm.at[pl.ds((start + j0) * _CH, _CH)], msg0, sm0)
            m1 = pltpu.async_copy(msg_hbm.at[pl.ds((start + j1) * _CH, _CH)], msg1, sm1)
            m0.wait()
            pltpu.sync_copy(msg0, agg_sh.at[idx2d_v.at[doff + j0]], add=True)
            m1.wait()
            pltpu.sync_copy(msg1, agg_sh.at[idx2d_v.at[doff + j1]], add=True)
            return carry

        lax.fori_loop(0, wch // 2, body, 0)

        if wch % 2:
            jl = wch - 1
            ml = pltpu.async_copy(msg_hbm.at[pl.ds((start + jl) * _CH, _CH)], msg0, sm0)
            ml.wait()
            pltpu.sync_copy(msg0, agg_sh.at[idx2d_v.at[doff + jl]], add=True)

        @pl.when(wid < xtra)
        def _xtra_scatter():
            b = (_NW * wch + wid) * _CH
            mt = pltpu.async_copy(msg_hbm.at[pl.ds(b, _CH)], msg1, sm1)
            mt.wait()
            pltpu.sync_copy(msg1, agg_sh.at[idx2d_v.at[wcha + exdoff + wid]], add=True)

        plsc.subcore_barrier()
        pltpu.sync_copy(agg_sh.at[pl.ds(rbase, _RPT)],
                        out_hbm.at[pl.ds(c * N + rbase, _RPT)])

        @pl.when(s == _NS - 1)
        def _out_tail():
            pltpu.sync_copy(agg_sh.at[pl.ds(_NS * _RPT, _RTAIL)],
                            out_hbm.at[pl.ds(c * N + _NS * _RPT, _RTAIL)])

    kern = pl.kernel(
        body_fn,
        out_type=jax.ShapeDtypeStruct((_NC * N, H), jnp.float32),
        mesh=plsc.VectorSubcoreMesh(core_axis_name="c", subcore_axis_name="s"),
        scratch_types=[
            pltpu.VMEM((wcha + max(xw, 8), _CH), jnp.int32),
            pltpu.VMEM((_CH, H), jnp.float32),
            pltpu.VMEM((_CH, H), jnp.float32),
            pltpu.VMEM_SHARED((N, H), jnp.float32),
            pltpu.SemaphoreType.DMA,
            pltpu.SemaphoreType.DMA,
        ],
    )
    return kern, r2d


_scatters = {ne: _make_scatter(ne // _CH) for ne in set(_PARTS)}


def _node_body(x_ref, a0_ref, a1_ref, w1a_ref, w1b_ref, b1_ref,
               w2_ref, b2_ref, w3_ref, b3_ref, w4_ref, b4_ref, out_ref):
    agg = a0_ref[...] + a1_ref[...]
    h = (jnp.dot(x_ref[...], w1a_ref[...], preferred_element_type=jnp.float32)
         + jnp.dot(agg, w1b_ref[...], preferred_element_type=jnp.float32)
         + b1_ref[...])
    h = jnp.maximum(h, 0.0)
    h = jnp.maximum(jnp.dot(h, w2_ref[...], preferred_element_type=jnp.float32) + b2_ref[...], 0.0)
    h = jnp.maximum(jnp.dot(h, w3_ref[...], preferred_element_type=jnp.float32) + b3_ref[...], 0.0)
    out_ref[...] = jnp.dot(h, w4_ref[...], preferred_element_type=jnp.float32) + b4_ref[...]


def _node_mlp(x, a0, a1, cw1a, cw1b, cb1, cw2, cb2, cw3, cb3, cw4, cb4):
    wb = lambda i: (0, 0)
    return pl.pallas_call(
        _node_body,
        grid=(N // _BN,),
        in_specs=[
            pl.BlockSpec((_BN, D), lambda i: (i, 0)),
            pl.BlockSpec((_BN, H), lambda i: (i, 0)),
            pl.BlockSpec((_BN, H), lambda i: (i, 0)),
            pl.BlockSpec((D, H), wb),
            pl.BlockSpec((H, H), wb),
            pl.BlockSpec((1, H), wb),
            pl.BlockSpec((H, H), wb),
            pl.BlockSpec((1, H), wb),
            pl.BlockSpec((H, H), wb),
            pl.BlockSpec((1, H), wb),
            pl.BlockSpec((H, D), wb),
            pl.BlockSpec((1, D), wb),
        ],
        out_specs=pl.BlockSpec((_BN, D), lambda i: (i, 0)),
        out_shape=jax.ShapeDtypeStruct((N, D), jnp.float32),
    )(x, a0, a1, cw1a, cw1b, cb1, cw2, cb2, cw3, cb3, cw4, cb4)


def kernel(x, edge_index, mw1, mb1, mw2, mb2, mw3, mb3,
           cw1, cb1, cw2, cb2, cw3, cb3, cw4, cb4):
    row = edge_index[0]
    col = edge_index[1]

    w1a = mw1[:D]
    w1b = mw1[D:]
    z5 = jnp.zeros((5, H), jnp.float32)
    wrel = jnp.concatenate([w1b[0:3], z5], axis=0)
    wdist = w1b[3:4]
    wunit = jnp.concatenate([w1b[4:7], z5], axis=0)

    t = _prep(x, w1a, mb1.reshape(1, H))
    xs, ys, zs = x[:, 0], x[:, 1], x[:, 2]
    rel8 = _rel(xs, ys, zs, row, col)
    mw2b = mb2.reshape(1, H)
    mw3b = mb3.reshape(1, H)

    parts = []
    off = 0
    for ne in _PARTS:
        rh = lax.slice(row, (off,), (off + ne,))
        ob = off // _BE
        off += ne
        tr = _gathers[ne](t, rh)
        msg = _edge_mlp(tr, rel8, ob, wrel, wdist, wunit, mw2, mw2b, mw3, mw3b)
        nch = ne // _CH
        r2d_rows = _scatters[ne][1]
        r2d = jnp.pad(rh.reshape(nch, _CH), ((0, r2d_rows - nch), (0, 0)))
        parts.append((ne, msg, r2d))

    acc = jnp.zeros((_NC * N, H), jnp.float32)
    for ne, msg, r2d in parts:
        acc = _scatters[ne][0](msg, r2d, acc)

    return _node_mlp(x, acc[:N], acc[N:], cw1[:D], cw1[D:],
                     cb1.reshape(1, H), cw2, cb2.reshape(1, H),
                     cw3, cb3.reshape(1, H), cw4, cb4.reshape(1, D))
